# baseline (speedup 1.0000x reference)
import functools

import jax
import jax.numpy as jnp
from jax import lax
from jax.experimental import pallas as pl
from jax.experimental.pallas import tpu as pltpu
from jax.experimental.pallas import tpu_sc as plsc

N_NODES = 50000
EMB_D = 64
HALF = 32
N_EDGES = 800000

NC = 2
NS = 16
NPAD = 50176
RPT = NPAD // NS
EPAD = 802816
CHUNK = 128
CROWS = CHUNK // 128
ZROWS = 112
DROWS = RPT // HALF
SINK = N_NODES + 8

_mesh = plsc.VectorSubcoreMesh(core_axis_name="c", subcore_axis_name="s")


def _fill(ref, n_words, value_vec):
    def body(i, _):
        ref[pl.ds(i * 16, 16)] = value_vec
        return 0
    lax.fori_loop(0, n_words // 16, body, 0)


@functools.partial(
    pl.kernel,
    out_type=jax.ShapeDtypeStruct((NC * NPAD,), jnp.float32),
    mesh=_mesh,
    scratch_types=[
        pltpu.VMEM((4, 128), jnp.int32),
        pltpu.VMEM((128,), jnp.float32),
        pltpu.VMEM((RPT,), jnp.float32),
        pltpu.VMEM_SHARED((NPAD,), jnp.float32),
        pltpu.SemaphoreType.DMA,
    ],
    compiler_params=pltpu.CompilerParams(use_tc_tiling_on_sc=False),
)
def _deg_kernel(dst_hbm, deg_out, idx_v, ones_v, stage_v, deg_sh, semh):
    c = lax.axis_index("c")
    s = lax.axis_index("s")
    ones16 = jnp.ones((16,), jnp.float32)
    for i in range(8):
        ones_v[pl.ds(i * 16, 16)] = ones16
    _fill(stage_v, RPT, jnp.zeros((16,), jnp.float32))
    pltpu.sync_copy(stage_v, deg_sh.at[pl.ds(s * RPT, RPT)])
    plsc.subcore_barrier()

    base_row = (c * NS + s) * (EPAD // (NC * NS) // 128)

    def chunk_body(k, _):
        pltpu.sync_copy(dst_hbm.at[pl.ds(base_row + k * 4, 4)], idx_v)
        for j in range(4):
            pltpu.async_copy(ones_v, deg_sh.at[idx_v.at[j]], semh, add=True)
        for j in range(4):
            pltpu.make_async_copy(ones_v, deg_sh.at[idx_v.at[j]],
                                  semh).wait()
        return 0

    lax.fori_loop(0, EPAD // (NC * NS) // 512, chunk_body, 0)
    plsc.subcore_barrier()
    pltpu.sync_copy(deg_sh.at[pl.ds(s * RPT, RPT)], stage_v)
    pltpu.sync_copy(stage_v, deg_out.at[pl.ds(c * NPAD + s * RPT, RPT)])


def _agg_body(src_hbm, dst_hbm, degp_hbm, ya_hbm, yb_hbm,
              y1a_out, y1b_out, suma_out, sumb_out,
              isrc0, isrc1, isrc2, isrc3, isrc4, isrc5, isrc6, isrc7,
              idst0, idst1, idst2, idst3, idst4, idst5, idst6, idst7,
              rows0, rows1, rows2, rows3, recip, acc_sh,
              semg, sems0, sems1, sems2, sems3,
              semi0, semi1, semi2, semi3, semi4, semi5, semi6, semi7):
    c = lax.axis_index("c")
    s = lax.axis_index("s")
    isrc = (isrc0, isrc1, isrc2, isrc3, isrc4, isrc5, isrc6, isrc7)
    idst = (idst0, idst1, idst2, idst3, idst4, idst5, idst6, idst7)
    rows = (rows0, rows1, rows2, rows3)
    sems = (sems0, sems1, sems2, sems3)
    semi = (semi0, semi1, semi2, semi3, semi4, semi5, semi6, semi7)

    pltpu.sync_copy(degp_hbm.at[pl.ds(s * DROWS, DROWS)],
                    rows0.at[pl.ds(0, DROWS)])
    pltpu.sync_copy(degp_hbm.at[pl.ds(NPAD // HALF + s * DROWS, DROWS)],
                    rows1.at[pl.ds(0, DROWS)])

    def rbody(i, _):
        a = rows0[i >> 1, pl.ds((i & 1) * 16, 16)]
        b = rows1[i >> 1, pl.ds((i & 1) * 16, 16)]
        d = a + b
        nz = jnp.where(d > 0.0, 1.0, 0.0)
        recip[pl.ds(i * 16, 16)] = nz / jnp.maximum(d, 1.0)
        return 0
    lax.fori_loop(0, RPT // 16, rbody, 0)

    z16 = jnp.zeros((16,), jnp.float32)

    def zbody(i, _):
        rows0[i >> 1, pl.ds((i & 1) * 16, 16)] = z16
        return 0
    lax.fori_loop(0, CHUNK * HALF // 16, zbody, 0)
    for q in range(RPT // ZROWS):
        pltpu.sync_copy(rows0.at[pl.ds(0, ZROWS)],
                        acc_sh.at[pl.ds(s * RPT + q * ZROWS, ZROWS)])
    plsc.subcore_barrier()

    idx_rows_per_tile = EPAD // NS // 128
    n_chunks = EPAD // NS // CHUNK

    def run(y_hbm):
        def fire_idx(k, m):
            r0 = s * idx_rows_per_tile + k * CROWS
            pltpu.async_copy(src_hbm.at[pl.ds(r0, CROWS)], isrc[m], semi[m])
            pltpu.async_copy(dst_hbm.at[pl.ds(r0, CROWS)], idst[m], semi[m])

        def drain_idx(k, m):
            r0 = s * idx_rows_per_tile + k * CROWS
            pltpu.make_async_copy(src_hbm.at[pl.ds(r0, CROWS)],
                                  isrc[m], semi[m]).wait()
            pltpu.make_async_copy(dst_hbm.at[pl.ds(r0, CROWS)],
                                  idst[m], semi[m]).wait()

        def fire_gathers(m, b):
            for j in range(CROWS):
                pltpu.async_copy(y_hbm.at[isrc[m].at[j]],
                                 rows[b].at[pl.ds(j * 128, 128)], semg)

        def drain_gathers(m, b):
            for j in range(CROWS):
                pltpu.make_async_copy(
                    y_hbm.at[isrc[m].at[j]],
                    rows[b].at[pl.ds(j * 128, 128)], semg).wait()

        def fire_scatters(m, b):
            for j in range(CROWS):
                pltpu.async_copy(rows[b].at[pl.ds(j * 128, 128)],
                                 acc_sh.at[idst[m].at[j]], sems[b], add=True)

        def drain_scatters(m, b):
            for j in range(CROWS):
                pltpu.make_async_copy(
                    rows[b].at[pl.ds(j * 128, 128)],
                    acc_sh.at[idst[m].at[j]], sems[b]).wait()

        for m in range(4):
            fire_idx(m, m)
        for j in range(8):
            if j >= 4:
                drain_scatters(j % 4, j % 4)
            fire_idx(j + 4, (j + 4) % 8)
            drain_idx(j, j)
            fire_gathers(j, j % 4)
            drain_gathers(j, j % 4)
            fire_scatters(j, j % 4)

        def oct_body(p, _):
            for j in range(8):
                k = 8 * p + j
                drain_scatters(j, j % 4)
                @pl.when(k + 4 < n_chunks)
                def _():
                    fire_idx(k + 4, (j + 4) % 8)
                drain_idx(k, j)
                fire_gathers(j, j % 4)
                drain_gathers(j, j % 4)
                fire_scatters(j, j % 4)
            return 0

        lax.fori_loop(1, n_chunks // 8, oct_body, 0)
        for b in range(4):
            drain_scatters(b, b)
        plsc.subcore_barrier()

    def scale_and_emit(y_out):
        for q in range(RPT // ZROWS):
            r0 = s * RPT + q * ZROWS
            pltpu.sync_copy(acc_sh.at[pl.ds(r0, ZROWS)],
                            rows0.at[pl.ds(0, ZROWS)])

            def sgrp(g, _):
                rv16 = recip[pl.ds(q * ZROWS + g * 16, 16)]
                for i in range(16):
                    r = g * 16 + i
                    v = jnp.full((16,), rv16[i], jnp.float32)
                    rows0[r, pl.ds(0, 16)] = rows0[r, pl.ds(0, 16)] * v
                    rows0[r, pl.ds(16, 16)] = rows0[r, pl.ds(16, 16)] * v
                return 0
            lax.fori_loop(0, ZROWS // 16, sgrp, 0)
            pltpu.sync_copy(rows0.at[pl.ds(0, ZROWS)],
                            y_out.at[pl.ds(r0, ZROWS)])
        plsc.subcore_barrier()

    def emit_sum(sum_out):
        for q in range(RPT // ZROWS):
            r0 = s * RPT + q * ZROWS
            pltpu.sync_copy(acc_sh.at[pl.ds(r0, ZROWS)],
                            rows0.at[pl.ds(0, ZROWS)])
            pltpu.sync_copy(rows0.at[pl.ds(0, ZROWS)],
                            sum_out.at[pl.ds(r0, ZROWS)])

    @pl.when(c == 0)
    def _():
        run(ya_hbm)
        scale_and_emit(y1a_out)
        run(y1a_out)
        emit_sum(suma_out)

    @pl.when(c == 1)
    def _():
        run(yb_hbm)
        scale_and_emit(y1b_out)
        run(y1b_out)
        emit_sum(sumb_out)


_agg = pl.kernel(
    _agg_body,
    out_type=[jax.ShapeDtypeStruct((NPAD, HALF), jnp.float32)] * 4,
    mesh=_mesh,
    scratch_types=(
        [pltpu.VMEM((CROWS, 128), jnp.int32)] * 16
        + [pltpu.VMEM((CHUNK, HALF), jnp.float32)] * 4
        + [pltpu.VMEM((RPT,), jnp.float32)]
        + [pltpu.VMEM_SHARED((NPAD, HALF), jnp.float32)]
        + [pltpu.SemaphoreType.DMA] * 13
    ),
    compiler_params=pltpu.CompilerParams(use_tc_tiling_on_sc=False),
)


BR = 3136
_TCGRID = NPAD // BR


def _rowspec(cols):
    return pl.BlockSpec((BR, cols), lambda i: (i, 0))


def _prep_body(deg_ref, emb_ref, dis_ref, ya_ref, yb_ref):
    deg = deg_ref[:, 0:1] + deg_ref[:, 1:2]
    dis = jnp.where(deg > 0.0, lax.rsqrt(jnp.maximum(deg, 1.0)), 0.0)
    dis_ref[...] = dis
    ya_ref[...] = emb_ref[:, :HALF] * dis
    yb_ref[...] = emb_ref[:, HALF:] * dis


_prep = pl.pallas_call(
    _prep_body,
    grid=(_TCGRID,),
    in_specs=[_rowspec(NC), _rowspec(EMB_D)],
    out_specs=[_rowspec(1), _rowspec(HALF), _rowspec(HALF)],
    out_shape=[jax.ShapeDtypeStruct((NPAD, 1), jnp.float32),
               jax.ShapeDtypeStruct((NPAD, HALF), jnp.float32),
               jax.ShapeDtypeStruct((NPAD, HALF), jnp.float32)],
)


def _final_body(emb_ref, sa_ref, sb_ref, dis_ref, out_ref):
    dis = dis_ref[...]
    third = jnp.float32(1.0 / 3.0)
    out_ref[:, :HALF] = (emb_ref[:, :HALF] + dis * sa_ref[...]) * third
    out_ref[:, HALF:] = (emb_ref[:, HALF:] + dis * sb_ref[...]) * third


_final = pl.pallas_call(
    _final_body,
    grid=(_TCGRID,),
    in_specs=[_rowspec(EMB_D), _rowspec(HALF), _rowspec(HALF), _rowspec(1)],
    out_specs=_rowspec(EMB_D),
    out_shape=jax.ShapeDtypeStruct((N_NODES, EMB_D), jnp.float32),
)


def kernel(edge_index, emb_weight):
    src = edge_index[0]
    dst = edge_index[1]
    pad = jnp.full((EPAD - N_EDGES,), SINK, jnp.int32)
    src_p = jnp.concatenate([src, pad]).reshape(EPAD // 128, 128)
    dst_p = jnp.concatenate([dst, pad]).reshape(EPAD // 128, 128)

    degp = _deg_kernel(dst_p)
    dis, y0a, y0b = _prep(degp.reshape(NC, NPAD).T, emb_weight)
    degp2d = degp.reshape(NC * NPAD // HALF, HALF)
    _y1a, _y1b, suma, sumb = _agg(src_p, dst_p, degp2d, y0a, y0b)
    return _final(emb_weight, suma, sumb, dis)


# speedup vs baseline: 1.0219x; 1.0219x over previous
import functools

import jax
import jax.numpy as jnp
from jax import lax
from jax.experimental import pallas as pl
from jax.experimental.pallas import tpu as pltpu
from jax.experimental.pallas import tpu_sc as plsc

N_NODES = 50000
EMB_D = 64
HALF = 32
N_EDGES = 800000

NC = 2
NS = 16
NPAD = 50176
RPT = NPAD // NS
EPAD = 811008
CHUNK = 384
CROWS = CHUNK // 128
ZROWS = 224
DROWS = RPT // HALF
SINK = N_NODES + 8

_mesh = plsc.VectorSubcoreMesh(core_axis_name="c", subcore_axis_name="s")


def _fill(ref, n_words, value_vec):
    def body(i, _):
        ref[pl.ds(i * 16, 16)] = value_vec
        return 0
    lax.fori_loop(0, n_words // 16, body, 0)


@functools.partial(
    pl.kernel,
    out_type=jax.ShapeDtypeStruct((NC * NPAD,), jnp.float32),
    mesh=_mesh,
    scratch_types=[
        pltpu.VMEM((CROWS, 128), jnp.int32),
        pltpu.VMEM((128,), jnp.float32),
        pltpu.VMEM((RPT,), jnp.float32),
        pltpu.VMEM_SHARED((NPAD,), jnp.float32),
        pltpu.SemaphoreType.DMA,
    ],
    compiler_params=pltpu.CompilerParams(use_tc_tiling_on_sc=False),
)
def _deg_kernel(dst_hbm, deg_out, idx_v, ones_v, stage_v, deg_sh, semh):
    c = lax.axis_index("c")
    s = lax.axis_index("s")
    ones16 = jnp.ones((16,), jnp.float32)
    for i in range(8):
        ones_v[pl.ds(i * 16, 16)] = ones16
    _fill(stage_v, RPT, jnp.zeros((16,), jnp.float32))
    pltpu.sync_copy(stage_v, deg_sh.at[pl.ds(s * RPT, RPT)])
    plsc.subcore_barrier()

    base_row = (c * NS + s) * (EPAD // (NC * NS) // 128)

    def chunk_body(k, _):
        pltpu.sync_copy(dst_hbm.at[pl.ds(base_row + k * CROWS, CROWS)], idx_v)
        for j in range(CROWS):
            pltpu.async_copy(ones_v, deg_sh.at[idx_v.at[j]], semh, add=True)
        for j in range(CROWS):
            pltpu.make_async_copy(ones_v, deg_sh.at[idx_v.at[j]],
                                  semh).wait()
        return 0

    lax.fori_loop(0, EPAD // (NC * NS) // CHUNK, chunk_body, 0)
    plsc.subcore_barrier()
    pltpu.sync_copy(deg_sh.at[pl.ds(s * RPT, RPT)], stage_v)
    pltpu.sync_copy(stage_v, deg_out.at[pl.ds(c * NPAD + s * RPT, RPT)])


def _agg_body(src_hbm, dst_hbm, degp_hbm, ya_hbm, yb_hbm,
              y1a_out, y1b_out, suma_out, sumb_out,
              isrc0, isrc1, isrc2, idst0, idst1, idst2,
              rows0, rows1, recip, acc_sh,
              semg, sems0, sems1, semi0, semi1, semi2):
    c = lax.axis_index("c")
    s = lax.axis_index("s")
    isrc = (isrc0, isrc1, isrc2)
    idst = (idst0, idst1, idst2)
    rows = (rows0, rows1)
    sems = (sems0, sems1)
    semi = (semi0, semi1, semi2)

    pltpu.sync_copy(degp_hbm.at[pl.ds(s * DROWS, DROWS)],
                    rows0.at[pl.ds(0, DROWS)])
    pltpu.sync_copy(degp_hbm.at[pl.ds(NPAD // HALF + s * DROWS, DROWS)],
                    rows1.at[pl.ds(0, DROWS)])

    def rbody(i, _):
        a = rows0[i >> 1, pl.ds((i & 1) * 16, 16)]
        b = rows1[i >> 1, pl.ds((i & 1) * 16, 16)]
        d = a + b
        nz = jnp.where(d > 0.0, 1.0, 0.0)
        recip[pl.ds(i * 16, 16)] = nz / jnp.maximum(d, 1.0)
        return 0
    lax.fori_loop(0, RPT // 16, rbody, 0)

    z16 = jnp.zeros((16,), jnp.float32)

    def zbody(i, _):
        rows0[i >> 1, pl.ds((i & 1) * 16, 16)] = z16
        return 0
    lax.fori_loop(0, CHUNK * HALF // 16, zbody, 0)
    for q in range(RPT // ZROWS):
        pltpu.sync_copy(rows0.at[pl.ds(0, ZROWS)],
                        acc_sh.at[pl.ds(s * RPT + q * ZROWS, ZROWS)])
    plsc.subcore_barrier()

    idx_rows_per_tile = EPAD // NS // 128
    n_chunks = EPAD // NS // CHUNK

    def run(y_hbm):
        def fire_idx(k, m):
            r0 = s * idx_rows_per_tile + k * CROWS
            pltpu.async_copy(src_hbm.at[pl.ds(r0, CROWS)], isrc[m], semi[m])
            pltpu.async_copy(dst_hbm.at[pl.ds(r0, CROWS)], idst[m], semi[m])

        def drain_idx(k, m):
            r0 = s * idx_rows_per_tile + k * CROWS
            pltpu.make_async_copy(src_hbm.at[pl.ds(r0, CROWS)],
                                  isrc[m], semi[m]).wait()
            pltpu.make_async_copy(dst_hbm.at[pl.ds(r0, CROWS)],
                                  idst[m], semi[m]).wait()

        def fire_gathers(m, b):
            for j in range(CROWS):
                pltpu.async_copy(y_hbm.at[isrc[m].at[j]],
                                 rows[b].at[pl.ds(j * 128, 128)], semg)

        def drain_gathers(m, b):
            for j in range(CROWS):
                pltpu.make_async_copy(
                    y_hbm.at[isrc[m].at[j]],
                    rows[b].at[pl.ds(j * 128, 128)], semg).wait()

        def fire_scatters(m, b):
            for j in range(CROWS):
                pltpu.async_copy(rows[b].at[pl.ds(j * 128, 128)],
                                 acc_sh.at[idst[m].at[j]], sems[b], add=True)

        def drain_scatters(m, b):
            for j in range(CROWS):
                pltpu.make_async_copy(
                    rows[b].at[pl.ds(j * 128, 128)],
                    acc_sh.at[idst[m].at[j]], sems[b]).wait()

        for m in range(3):
            fire_idx(m, m)
        for j in range(6):
            if j >= 2:
                drain_scatters(j % 3, j % 2)
                fire_idx(j + 1, (j + 1) % 3)
            drain_idx(j, j % 3)
            fire_gathers(j % 3, j % 2)
            drain_gathers(j % 3, j % 2)
            fire_scatters(j % 3, j % 2)

        def six_body(p, _):
            for j in range(6):
                k = 6 * p + j
                drain_scatters((j + 1) % 3, j % 2)
                @pl.when(k + 1 < n_chunks)
                def _():
                    fire_idx(k + 1, (j + 1) % 3)
                drain_idx(k, j % 3)
                fire_gathers(j % 3, j % 2)
                drain_gathers(j % 3, j % 2)
                fire_scatters(j % 3, j % 2)
            return 0

        lax.fori_loop(1, n_chunks // 6, six_body, 0)
        drain_scatters(1, 0)
        drain_scatters(2, 1)
        plsc.subcore_barrier()

    def scale_and_emit(y_out):
        for q in range(RPT // ZROWS):
            r0 = s * RPT + q * ZROWS
            pltpu.sync_copy(acc_sh.at[pl.ds(r0, ZROWS)],
                            rows0.at[pl.ds(0, ZROWS)])

            def sgrp(g, _):
                rv16 = recip[pl.ds(q * ZROWS + g * 16, 16)]
                for i in range(16):
                    r = g * 16 + i
                    v = jnp.full((16,), rv16[i], jnp.float32)
                    rows0[r, pl.ds(0, 16)] = rows0[r, pl.ds(0, 16)] * v
                    rows0[r, pl.ds(16, 16)] = rows0[r, pl.ds(16, 16)] * v
                return 0
            lax.fori_loop(0, ZROWS // 16, sgrp, 0)
            pltpu.sync_copy(rows0.at[pl.ds(0, ZROWS)],
                            y_out.at[pl.ds(r0, ZROWS)])
        plsc.subcore_barrier()

    def emit_sum(sum_out):
        for q in range(RPT // ZROWS):
            r0 = s * RPT + q * ZROWS
            pltpu.sync_copy(acc_sh.at[pl.ds(r0, ZROWS)],
                            rows0.at[pl.ds(0, ZROWS)])
            pltpu.sync_copy(rows0.at[pl.ds(0, ZROWS)],
                            sum_out.at[pl.ds(r0, ZROWS)])

    @pl.when(c == 0)
    def _():
        run(ya_hbm)
        scale_and_emit(y1a_out)
        run(y1a_out)
        emit_sum(suma_out)

    @pl.when(c == 1)
    def _():
        run(yb_hbm)
        scale_and_emit(y1b_out)
        run(y1b_out)
        emit_sum(sumb_out)


_agg = pl.kernel(
    _agg_body,
    out_type=[jax.ShapeDtypeStruct((NPAD, HALF), jnp.float32)] * 4,
    mesh=_mesh,
    scratch_types=(
        [pltpu.VMEM((CROWS, 128), jnp.int32)] * 6
        + [pltpu.VMEM((CHUNK, HALF), jnp.float32)] * 2
        + [pltpu.VMEM((RPT,), jnp.float32)]
        + [pltpu.VMEM_SHARED((NPAD, HALF), jnp.float32)]
        + [pltpu.SemaphoreType.DMA] * 6
    ),
    compiler_params=pltpu.CompilerParams(use_tc_tiling_on_sc=False),
)


BR = 3136
_TCGRID = NPAD // BR


def _rowspec(cols):
    return pl.BlockSpec((BR, cols), lambda i: (i, 0))


def _prep_body(deg_ref, emb_ref, dis_ref, ya_ref, yb_ref):
    deg = deg_ref[:, 0:1] + deg_ref[:, 1:2]
    dis = jnp.where(deg > 0.0, lax.rsqrt(jnp.maximum(deg, 1.0)), 0.0)
    dis_ref[...] = dis
    ya_ref[...] = emb_ref[:, :HALF] * dis
    yb_ref[...] = emb_ref[:, HALF:] * dis


_prep = pl.pallas_call(
    _prep_body,
    grid=(_TCGRID,),
    in_specs=[_rowspec(NC), _rowspec(EMB_D)],
    out_specs=[_rowspec(1), _rowspec(HALF), _rowspec(HALF)],
    out_shape=[jax.ShapeDtypeStruct((NPAD, 1), jnp.float32),
               jax.ShapeDtypeStruct((NPAD, HALF), jnp.float32),
               jax.ShapeDtypeStruct((NPAD, HALF), jnp.float32)],
)


def _final_body(emb_ref, sa_ref, sb_ref, dis_ref, out_ref):
    dis = dis_ref[...]
    third = jnp.float32(1.0 / 3.0)
    out_ref[:, :HALF] = (emb_ref[:, :HALF] + dis * sa_ref[...]) * third
    out_ref[:, HALF:] = (emb_ref[:, HALF:] + dis * sb_ref[...]) * third


_final = pl.pallas_call(
    _final_body,
    grid=(_TCGRID,),
    in_specs=[_rowspec(EMB_D), _rowspec(HALF), _rowspec(HALF), _rowspec(1)],
    out_specs=_rowspec(EMB_D),
    out_shape=jax.ShapeDtypeStruct((N_NODES, EMB_D), jnp.float32),
)


def kernel(edge_index, emb_weight):
    src = edge_index[0]
    dst = edge_index[1]
    pad = jnp.full((EPAD - N_EDGES,), SINK, jnp.int32)
    src_p = jnp.concatenate([src, pad]).reshape(EPAD // 128, 128)
    dst_p = jnp.concatenate([dst, pad]).reshape(EPAD // 128, 128)

    degp = _deg_kernel(dst_p)
    dis, y0a, y0b = _prep(degp.reshape(NC, NPAD).T, emb_weight)
    degp2d = degp.reshape(NC * NPAD // HALF, HALF)
    _y1a, _y1b, suma, sumb = _agg(src_p, dst_p, degp2d, y0a, y0b)
    return _final(emb_weight, suma, sumb, dis)
